# baseline (speedup 1.0000x reference)
import jax
import jax.numpy as jnp
from jax.experimental import pallas as pl
from jax.experimental.pallas import tpu as pltpu

B = 1024
LAT = 64
ACTIONS = 64
POL_CH = 2
NCHAIN = 4
HHALF = 8


def _body(x_ref, wc_ref, bcb_ref, wfc_ref, bfc_ref, out_ref):
    j = pl.program_id(1)
    tiles = [None] * (POL_CH * HHALF * 2)
    for h in range(HHALF):
        for wh in range(2):
            a0 = [None] * NCHAIN
            a1 = [None] * NCHAIN
            for c in range(LAT):
                t = x_ref[c, h, wh, 0]
                p0 = t * wc_ref[0, c]
                p1 = t * wc_ref[1, c]
                k = c % NCHAIN
                a0[k] = p0 if a0[k] is None else a0[k] + p0
                a1[k] = p1 if a1[k] is None else a1[k] + p1
            s0 = (a0[0] + a0[1]) + (a0[2] + a0[3])
            s1 = (a1[0] + a1[1]) + (a1[2] + a1[3])
            tiles[h * 2 + wh] = jnp.maximum(s0 + bcb_ref[0], 0.0)
            tiles[HHALF * 2 + h * 2 + wh] = jnp.maximum(s1 + bcb_ref[1], 0.0)
    rhs = jnp.concatenate(tiles, axis=0)
    part = jnp.dot(wfc_ref[0], rhs, preferred_element_type=jnp.float32)

    @pl.when(j == 0)
    def _first():
        out_ref[...] = part + bfc_ref[...]

    @pl.when(j != 0)
    def _rest():
        out_ref[...] += part


def kernel(x, W_conv, b_conv, W_fc, b_fc):
    x6 = x.reshape(8, 128, LAT, 16, 2, 8).transpose(2, 3, 4, 0, 5, 1)
    bcb = jnp.broadcast_to(b_conv[:, None, None], (POL_CH, 8, 128))
    bfc_col = b_fc[:, None]
    wfs = (
        W_fc.reshape(ACTIONS, POL_CH, 2, HHALF, 16)
        .transpose(2, 0, 1, 3, 4)
        .reshape(2, ACTIONS, POL_CH * HHALF * 16)
    )

    out = pl.pallas_call(
        _body,
        grid=(8, 2),
        in_specs=[
            pl.BlockSpec((LAT, HHALF, 2, 1, 8, 128),
                         lambda i, j: (0, j, 0, i, 0, 0)),
            pl.BlockSpec(memory_space=pltpu.SMEM),
            pl.BlockSpec((POL_CH, 8, 128), lambda i, j: (0, 0, 0)),
            pl.BlockSpec((1, ACTIONS, POL_CH * HHALF * 16),
                         lambda i, j: (j, 0, 0)),
            pl.BlockSpec((ACTIONS, 1), lambda i, j: (0, 0)),
        ],
        out_specs=pl.BlockSpec((ACTIONS, 128), lambda i, j: (0, i)),
        out_shape=jax.ShapeDtypeStruct((ACTIONS, B), jnp.float32),
        compiler_params=pltpu.CompilerParams(
            dimension_semantics=("arbitrary", "arbitrary"),
        ),
    )(x6, W_conv, bcb, wfs, bfc_col)
    return out.T


# speedup vs baseline: 1.2615x; 1.2615x over previous
import jax
import jax.numpy as jnp
from jax.experimental import pallas as pl
from jax.experimental.pallas import tpu as pltpu

B = 1024
LAT = 64
ACTIONS = 64
POL_CH = 2
HQ = 4


def _body(x_ref, wc_ref, bcb_ref, wfc_ref, bfc_ref, out_ref):
    tiles = [None] * (POL_CH * 32)
    for q in range(16 // HQ):
        acc0 = None
        acc1 = None
        for c in range(LAT):
            t = x_ref[c, q * HQ:(q + 1) * HQ, :, 0]
            p0 = t * wc_ref[0, c]
            p1 = t * wc_ref[1, c]
            acc0 = p0 if acc0 is None else acc0 + p0
            acc1 = p1 if acc1 is None else acc1 + p1
        r0 = jnp.maximum(acc0 + bcb_ref[0], 0.0)
        r1 = jnp.maximum(acc1 + bcb_ref[1], 0.0)
        for hl in range(HQ):
            for wh in range(2):
                h = q * HQ + hl
                tiles[h * 2 + wh] = r0[hl, wh]
                tiles[32 + h * 2 + wh] = r1[hl, wh]
    rhs = jnp.concatenate(tiles, axis=0)
    out_ref[...] = (
        jnp.dot(wfc_ref[...], rhs, preferred_element_type=jnp.float32)
        + bfc_ref[...]
    )


def kernel(x, W_conv, b_conv, W_fc, b_fc):
    x6 = x.reshape(8, 128, LAT, 16, 2, 8).transpose(2, 3, 4, 0, 5, 1)
    bcb = jnp.broadcast_to(
        b_conv[:, None, None, None, None], (POL_CH, 1, 1, 8, 128)
    )
    bfc_col = b_fc[:, None]

    out = pl.pallas_call(
        _body,
        grid=(8,),
        in_specs=[
            pl.BlockSpec((LAT, 16, 2, 1, 8, 128), lambda i: (0, 0, 0, i, 0, 0)),
            pl.BlockSpec(memory_space=pltpu.SMEM),
            pl.BlockSpec((POL_CH, 1, 1, 8, 128), lambda i: (0, 0, 0, 0, 0)),
            pl.BlockSpec((ACTIONS, POL_CH * 256), lambda i: (0, 0)),
            pl.BlockSpec((ACTIONS, 1), lambda i: (0, 0)),
        ],
        out_specs=pl.BlockSpec((ACTIONS, 128), lambda i: (0, i)),
        out_shape=jax.ShapeDtypeStruct((ACTIONS, B), jnp.float32),
        compiler_params=pltpu.CompilerParams(
            dimension_semantics=("arbitrary",),
        ),
    )(x6, W_conv, bcb, W_fc, bfc_col)
    return out.T
